# baseline (speedup 1.0000x reference)
import functools

import jax
import jax.numpy as jnp
from jax.experimental import pallas as pl
from jax.experimental.pallas import tpu as pltpu

_B = 2048


def _e2_kernel(cb_ref, e2_ref):
    for lvl in range(cb_ref.shape[0]):
        e2_ref[lvl, :] = jnp.sum(jnp.square(cb_ref[lvl]), axis=1)


def _rvq_kernel(x_ref, cb_ref, hi_ref, mid_ref, lo_ref, e2_ref, out_ref):
    num_levels, K, D = cb_ref.shape
    B = x_ref.shape[0]
    residual = x_ref[...]
    flat = residual
    q_sum = jnp.zeros_like(residual)
    iota = jax.lax.broadcasted_iota(jnp.int32, (B, K), 1)
    for lvl in range(num_levels):
        cb = cb_ref[lvl]
        r2 = jnp.sum(jnp.square(residual), axis=1, keepdims=True)
        sim = jax.lax.dot_general(
            residual, cb,
            dimension_numbers=(((1,), (1,)), ((), ())),
            preferred_element_type=jnp.float32,
        )
        dist = r2 + e2_ref[lvl, :] - 2.0 * sim
        mn = jnp.min(dist, axis=1, keepdims=True)
        idx = jnp.min(jnp.where(dist == mn, iota, K), axis=1, keepdims=True)
        onehot = (iota == idx).astype(jnp.bfloat16)

        def _pick(plane_ref):
            return jax.lax.dot_general(
                onehot, plane_ref[lvl],
                dimension_numbers=(((1,), (0,)), ((), ())),
                preferred_element_type=jnp.float32,
            )

        q = _pick(hi_ref) + (_pick(mid_ref) + _pick(lo_ref))
        q_sum = q_sum + q
        residual = residual - q
    out_ref[...] = flat + (q_sum - flat)


@functools.partial(jax.jit, static_argnames=())
def kernel(x, codebooks):
    shape = x.shape
    D = shape[-1]
    flat = jnp.reshape(x, (-1, D))
    N = flat.shape[0]
    M, K, _ = codebooks.shape

    def _trunc16(v):
        bits = jax.lax.bitcast_convert_type(v, jnp.uint32)
        return jax.lax.bitcast_convert_type(
            bits & jnp.uint32(0xFFFF0000), jnp.float32)

    hi_f = _trunc16(codebooks)
    rem1 = codebooks - hi_f
    mid_f = _trunc16(rem1)
    lo_f = rem1 - mid_f
    cb_hi = hi_f.astype(jnp.bfloat16)
    cb_mid = mid_f.astype(jnp.bfloat16)
    cb_lo = lo_f.astype(jnp.bfloat16)

    full = lambda a: pl.BlockSpec(a.shape, lambda i: (0,) * a.ndim)

    e2 = pl.pallas_call(
        _e2_kernel,
        in_specs=[pl.BlockSpec(codebooks.shape, lambda: (0, 0, 0))],
        out_specs=pl.BlockSpec((M, K), lambda: (0, 0)),
        out_shape=jax.ShapeDtypeStruct((M, K), jnp.float32),
    )(codebooks)

    out = pl.pallas_call(
        _rvq_kernel,
        grid=(N // _B,),
        in_specs=[
            pl.BlockSpec((_B, D), lambda i: (i, 0)),
            full(codebooks), full(cb_hi), full(cb_mid), full(cb_lo),
            full(e2),
        ],
        out_specs=pl.BlockSpec((_B, D), lambda i: (i, 0)),
        out_shape=jax.ShapeDtypeStruct((N, D), jnp.float32),
        compiler_params=pltpu.CompilerParams(
            dimension_semantics=("parallel",),
        ),
    )(flat, codebooks, cb_hi, cb_mid, cb_lo, e2)
    return jnp.reshape(out, shape)


# speedup vs baseline: 1.5217x; 1.5217x over previous
import functools

import jax
import jax.numpy as jnp
from jax.experimental import pallas as pl
from jax.experimental.pallas import tpu as pltpu

_B = 1024


def _e2_kernel(cb_ref, e2_ref):
    for lvl in range(cb_ref.shape[0]):
        e2_ref[lvl, :] = jnp.sum(jnp.square(cb_ref[lvl]), axis=1)


def _rvq_kernel(x_ref, cb_ref, hi_ref, mid_ref, lo_ref, e2_ref, out_ref):
    num_levels, K, D = cb_ref.shape
    B = x_ref.shape[0]
    H = B // 2
    res = [x_ref[0:H, :], x_ref[H:B, :]]
    flat = list(res)
    q_sum = [jnp.zeros_like(res[0]), jnp.zeros_like(res[1])]
    iota = jax.lax.broadcasted_iota(jnp.int32, (H, K), 1)
    for lvl in range(num_levels):
        cb = cb_ref[lvl]
        for h in (0, 1):
            r2 = jnp.sum(jnp.square(res[h]), axis=1, keepdims=True)
            sim = jax.lax.dot_general(
                res[h], cb,
                dimension_numbers=(((1,), (1,)), ((), ())),
                preferred_element_type=jnp.float32,
            )
            dist = r2 + e2_ref[lvl, :] - 2.0 * sim
            mn = jnp.min(dist, axis=1, keepdims=True)
            idx = jnp.min(jnp.where(dist == mn, iota, K), axis=1, keepdims=True)
            onehot = (iota == idx).astype(jnp.bfloat16)

            def _pick(plane_ref):
                return jax.lax.dot_general(
                    onehot, plane_ref[lvl],
                    dimension_numbers=(((1,), (0,)), ((), ())),
                    preferred_element_type=jnp.float32,
                )

            q = _pick(hi_ref) + (_pick(mid_ref) + _pick(lo_ref))
            q_sum[h] = q_sum[h] + q
            res[h] = res[h] - q
    out_ref[0:H, :] = flat[0] + (q_sum[0] - flat[0])
    out_ref[H:B, :] = flat[1] + (q_sum[1] - flat[1])


@functools.partial(jax.jit, static_argnames=())
def kernel(x, codebooks):
    shape = x.shape
    D = shape[-1]
    flat = jnp.reshape(x, (-1, D))
    N = flat.shape[0]
    M, K, _ = codebooks.shape

    def _trunc16(v):
        bits = jax.lax.bitcast_convert_type(v, jnp.uint32)
        return jax.lax.bitcast_convert_type(
            bits & jnp.uint32(0xFFFF0000), jnp.float32)

    hi_f = _trunc16(codebooks)
    rem1 = codebooks - hi_f
    mid_f = _trunc16(rem1)
    lo_f = rem1 - mid_f
    cb_hi = hi_f.astype(jnp.bfloat16)
    cb_mid = mid_f.astype(jnp.bfloat16)
    cb_lo = lo_f.astype(jnp.bfloat16)

    full = lambda a: pl.BlockSpec(a.shape, lambda i: (0,) * a.ndim)

    e2 = pl.pallas_call(
        _e2_kernel,
        in_specs=[pl.BlockSpec(codebooks.shape, lambda: (0, 0, 0))],
        out_specs=pl.BlockSpec((M, K), lambda: (0, 0)),
        out_shape=jax.ShapeDtypeStruct((M, K), jnp.float32),
    )(codebooks)

    out = pl.pallas_call(
        _rvq_kernel,
        grid=(N // _B,),
        in_specs=[
            pl.BlockSpec((_B, D), lambda i: (i, 0)),
            full(codebooks), full(cb_hi), full(cb_mid), full(cb_lo),
            full(e2),
        ],
        out_specs=pl.BlockSpec((_B, D), lambda i: (i, 0)),
        out_shape=jax.ShapeDtypeStruct((N, D), jnp.float32),
        compiler_params=pltpu.CompilerParams(
            dimension_semantics=("parallel",),
        ),
    )(flat, codebooks, cb_hi, cb_mid, cb_lo, e2)
    return jnp.reshape(out, shape)
